# Initial kernel scaffold; baseline (speedup 1.0000x reference)
import jax
import jax.numpy as jnp
from jax.experimental import pallas as pl


def kernel(x, edge_index, batch, edge_weight, W0, b0, W1, b1, W2, b2, gw0, gb0, ga0, gw1, gb1, ga1, gw2, gb2, ga2, Wi, bi, Wd0, bd0, Wd1, bd1, Wo, bo):
    raise NotImplementedError("write your pallas kernel here")


# speedup vs baseline: 8.1331x; 8.1331x over previous
import functools

import jax
import jax.numpy as jnp
from jax import lax
from jax.experimental import pallas as pl
from jax.experimental.pallas import tpu as pltpu
from jax.experimental.pallas import tpu_sc as plsc

NC = 2
NS = 16
L = 16
G = 16


def _mesh():
    return plsc.VectorSubcoreMesh(core_axis_name="c", subcore_axis_name="s")


@functools.lru_cache(maxsize=None)
def _sc_deg(n, e):
    ept = e // (NC * NS)
    ck = 80
    nchunks = ept // ck
    big = (n + NS - 1) // NS
    big = ((big + 7) // 8) * 8
    last = n - big * (NS - 1)

    @functools.partial(
        pl.kernel,
        out_type=jax.ShapeDtypeStruct((NC * n,), jnp.float32),
        mesh=_mesh(),
        scratch_types=[
            pltpu.VMEM_SHARED((n,), jnp.float32),
            pltpu.VMEM((big,), jnp.float32),
            pltpu.VMEM((ck,), jnp.int32),
            pltpu.VMEM((ck,), jnp.float32),
            pltpu.SemaphoreType.DMA,
        ],
    )
    def k(dst_hbm, w_hbm, out_hbm, acc, zbuf, dst_v, w_v, sem):
        cid = lax.axis_index("c")
        sid = lax.axis_index("s")
        wid = cid * NS + sid

        def zb(i, _):
            zbuf[pl.ds(i * L, L)] = jnp.zeros((L,), jnp.float32)
            return 0
        lax.fori_loop(0, big // L, zb, 0)

        @pl.when(sid < NS - 1)
        def _():
            pltpu.sync_copy(zbuf, acc.at[pl.ds(sid * big, big)])

        @pl.when(sid == NS - 1)
        def _():
            pltpu.sync_copy(zbuf.at[pl.ds(0, last)],
                            acc.at[pl.ds((NS - 1) * big, last)])

        plsc.subcore_barrier()

        def body(i, _):
            base = wid * ept + i * ck
            pltpu.sync_copy(dst_hbm.at[pl.ds(base, ck)], dst_v)
            pltpu.sync_copy(w_hbm.at[pl.ds(base, ck)], w_v)
            pltpu.async_copy(w_v, acc.at[dst_v], sem, add=True).wait()
            return 0
        lax.fori_loop(0, nchunks, body, 0)

        plsc.subcore_barrier()

        @pl.when(sid < NS - 1)
        def _():
            pltpu.sync_copy(acc.at[pl.ds(sid * big, big)], zbuf)
            pltpu.sync_copy(zbuf, out_hbm.at[pl.ds(cid * n + sid * big, big)])

        @pl.when(sid == NS - 1)
        def _():
            pltpu.sync_copy(acc.at[pl.ds((NS - 1) * big, last)],
                            zbuf.at[pl.ds(0, last)])
            pltpu.sync_copy(zbuf.at[pl.ds(0, last)],
                            out_hbm.at[pl.ds(cid * n + (NS - 1) * big, last)])

    return k


@functools.lru_cache(maxsize=None)
def _sc_msg(n, h, e):
    ept = e // (NC * NS)
    ck = 80
    nchunks = ept // ck
    bigr = ((n // NS + 7) // 8) * 8
    lastr = n - bigr * (NS - 1)

    def _sub_range(sid_static_last):
        total = lastr if sid_static_last else bigr
        parts, off = [], 0
        while off < total:
            cz = min(ck, total - off)
            parts.append((off, cz))
            off += cz
        return parts

    @functools.partial(
        pl.kernel,
        out_type=jax.ShapeDtypeStruct((NC * n, h), jnp.float32),
        mesh=_mesh(),
        scratch_types=[
            pltpu.VMEM_SHARED((n, h), jnp.float32),
            pltpu.VMEM((ck,), jnp.int32),
            pltpu.VMEM((ck,), jnp.int32),
            pltpu.VMEM((ck,), jnp.float32),
            pltpu.VMEM((ck, h), jnp.float32),
            pltpu.SemaphoreType.DMA,
        ],
    )
    def k(hp_hbm, src_hbm, dst_hbm, w_hbm, out_hbm,
          acc, src_v, dst_v, w_v, rows_v, sem):
        cid = lax.axis_index("c")
        sid = lax.axis_index("s")
        wid = cid * NS + sid

        def zr(i, _):
            r = i // (h // L)
            c = i % (h // L)
            rows_v[r, pl.ds(c * L, L)] = jnp.zeros((L,), jnp.float32)
            return 0
        lax.fori_loop(0, ck * (h // L), zr, 0)

        @pl.when(sid < NS - 1)
        def _():
            for off, cz in _sub_range(False):
                pltpu.sync_copy(rows_v.at[pl.ds(0, cz)],
                                acc.at[pl.ds(sid * bigr + off, cz)])

        @pl.when(sid == NS - 1)
        def _():
            for off, cz in _sub_range(True):
                pltpu.sync_copy(rows_v.at[pl.ds(0, cz)],
                                acc.at[pl.ds((NS - 1) * bigr + off, cz)])

        plsc.subcore_barrier()

        def body(i, _):
            base = wid * ept + i * ck
            pltpu.sync_copy(src_hbm.at[pl.ds(base, ck)], src_v)
            pltpu.sync_copy(dst_hbm.at[pl.ds(base, ck)], dst_v)
            pltpu.sync_copy(w_hbm.at[pl.ds(base, ck)], w_v)
            pltpu.async_copy(hp_hbm.at[src_v], rows_v, sem).wait()

            def sgrp(g, _):
                wg = w_v[pl.ds(g * L, L)]
                for j in range(L):
                    r = g * L + j
                    wsp = jnp.full((L,), wg[j], jnp.float32)
                    for c in range(h // L):
                        sl = pl.ds(c * L, L)
                        rows_v[r, sl] = rows_v[r, sl] * wsp
                return 0
            lax.fori_loop(0, ck // L, sgrp, 0)

            pltpu.async_copy(rows_v, acc.at[dst_v], sem, add=True).wait()
            return 0
        lax.fori_loop(0, nchunks, body, 0)

        plsc.subcore_barrier()

        @pl.when(sid < NS - 1)
        def _():
            for off, cz in _sub_range(False):
                base = sid * bigr + off
                pltpu.sync_copy(acc.at[pl.ds(base, cz)],
                                rows_v.at[pl.ds(0, cz)])
                pltpu.sync_copy(rows_v.at[pl.ds(0, cz)],
                                out_hbm.at[pl.ds(cid * n + base, cz)])

        @pl.when(sid == NS - 1)
        def _():
            for off, cz in _sub_range(True):
                base = (NS - 1) * bigr + off
                pltpu.sync_copy(acc.at[pl.ds(base, cz)],
                                rows_v.at[pl.ds(0, cz)])
                pltpu.sync_copy(rows_v.at[pl.ds(0, cz)],
                                out_hbm.at[pl.ds(cid * n + base, cz)])

    return k


BN = 1000


def _col(n, w):
    return pl.BlockSpec((BN, w), lambda i: (i, 0))


def _full(a, b):
    return pl.BlockSpec((a, b), lambda i: (0, 0))


@functools.lru_cache(maxsize=None)
def _tc_prep(n, d, h):
    def body(degt_ref, x_ref, w0_ref, dinv_ref, hp_ref):
        deg = jnp.sum(degt_ref[...], axis=1, keepdims=True) + 1.0
        dinv = lax.rsqrt(deg)
        dinv_ref[...] = dinv
        hw = jnp.dot(x_ref[...], w0_ref[...],
                     preferred_element_type=jnp.float32)
        hp_ref[...] = hw * dinv

    return pl.pallas_call(
        body,
        grid=(n // BN,),
        in_specs=[_col(n, 2), _col(n, d), _full(d, h)],
        out_specs=[_col(n, 1), _col(n, h)],
        out_shape=[jax.ShapeDtypeStruct((n, 1), jnp.float32),
                   jax.ShapeDtypeStruct((n, h), jnp.float32)],
    )


@functools.lru_cache(maxsize=None)
def _tc_post_a(n, h):
    def body(a0_ref, a1_ref, hp_ref, dinv_ref, b_ref, br_ref,
             pre_ref, ssum_ref, cnt_ref):
        i = pl.program_id(0)
        pre = ((a0_ref[...] + a1_ref[...] + hp_ref[...]) * dinv_ref[...]
               + b_ref[...])
        pre_ref[...] = pre
        oh = jnp.where(
            lax.broadcasted_iota(jnp.int32, (G, BN), 0) == br_ref[0],
            1.0, 0.0)

        @pl.when(i == 0)
        def _():
            ssum_ref[...] = jnp.zeros_like(ssum_ref)
            cnt_ref[...] = jnp.zeros_like(cnt_ref)

        ssum_ref[...] += jnp.dot(oh, pre, preferred_element_type=jnp.float32)
        cnt_ref[...] += jnp.sum(oh, axis=1, keepdims=True)

    return pl.pallas_call(
        body,
        grid=(n // BN,),
        in_specs=[_col(n, h), _col(n, h), _col(n, h), _col(n, 1),
                  _full(1, h), pl.BlockSpec((1, 1, BN), lambda i: (i, 0, 0))],
        out_specs=[_col(n, h), _full(G, h), _full(G, 1)],
        out_shape=[jax.ShapeDtypeStruct((n, h), jnp.float32),
                   jax.ShapeDtypeStruct((G, h), jnp.float32),
                   jax.ShapeDtypeStruct((G, 1), jnp.float32)],
    )


@functools.lru_cache(maxsize=None)
def _tc_post_b(n, h):
    def body(pre_ref, ssum_ref, cnt_ref, ga_ref, br_ref, bc_ref,
             ctr_ref, vsum_ref):
        i = pl.program_id(0)
        cnt = jnp.maximum(cnt_ref[...], 1.0)
        mean = ssum_ref[...] / cnt
        ohT = jnp.where(
            lax.broadcasted_iota(jnp.int32, (BN, G), 1) == bc_ref[...],
            1.0, 0.0)
        ctr = pre_ref[...] - ga_ref[...] * jnp.dot(
            ohT, mean, preferred_element_type=jnp.float32)
        ctr_ref[...] = ctr
        oh = jnp.where(
            lax.broadcasted_iota(jnp.int32, (G, BN), 0) == br_ref[0],
            1.0, 0.0)

        @pl.when(i == 0)
        def _():
            vsum_ref[...] = jnp.zeros_like(vsum_ref)

        vsum_ref[...] += jnp.dot(oh, ctr * ctr,
                                 preferred_element_type=jnp.float32)

    return pl.pallas_call(
        body,
        grid=(n // BN,),
        in_specs=[_col(n, h), _full(G, h), _full(G, 1), _full(1, h),
                  pl.BlockSpec((1, 1, BN), lambda i: (i, 0, 0)), _col(n, 1)],
        out_specs=[_col(n, h), _full(G, h)],
        out_shape=[jax.ShapeDtypeStruct((n, h), jnp.float32),
                   jax.ShapeDtypeStruct((G, h), jnp.float32)],
    )


@functools.lru_cache(maxsize=None)
def _tc_post_c(n, h, with_next, with_res):
    def body(*refs):
        it = iter(refs)
        ctr_ref = next(it)
        vsum_ref = next(it)
        cnt_ref = next(it)
        gw_ref = next(it)
        gb_ref = next(it)
        dinv_ref = next(it)
        bc_ref = next(it)
        wn_ref = next(it) if with_next else None
        hprev_ref = next(it) if with_res else None
        h_ref = next(it)
        flat_ref = next(it)
        hpn_ref = next(it) if with_next else None

        i = pl.program_id(0)
        cnt = jnp.maximum(cnt_ref[...], 1.0)
        den = jnp.sqrt(vsum_ref[...] / cnt + 1e-5)
        bc = bc_ref[...]
        ohT = jnp.where(
            lax.broadcasted_iota(jnp.int32, (BN, G), 1) == bc, 1.0, 0.0)
        hnew = jnp.maximum(
            gw_ref[...] * ctr_ref[...]
            / jnp.dot(ohT, den, preferred_element_type=jnp.float32)
            + gb_ref[...], 0.0)
        if with_res:
            hnew = hnew + hprev_ref[...]
        h_ref[...] = hnew

        neg = jnp.float32(-jnp.inf)
        rows = []
        for gg in range(G):
            rows.append(jnp.max(jnp.where(bc == gg, hnew, neg), axis=0,
                                keepdims=True))
        bmax = jnp.concatenate(rows, axis=0)

        @pl.when(i == 0)
        def _():
            flat_ref[...] = jnp.full_like(flat_ref, neg)

        flat_ref[...] = jnp.maximum(flat_ref[...], bmax)

        if with_next:
            hpn_ref[...] = jnp.dot(
                hnew, wn_ref[...],
                preferred_element_type=jnp.float32) * dinv_ref[...]

    in_specs = [_col(n, h), _full(G, h), _full(G, 1), _full(1, h),
                _full(1, h), _col(n, 1), _col(n, 1)]
    if with_next:
        in_specs.append(_full(h, h))
    if with_res:
        in_specs.append(_col(n, h))
    out_specs = [_col(n, h), _full(G, h)]
    outs = [jax.ShapeDtypeStruct((n, h), jnp.float32),
            jax.ShapeDtypeStruct((G, h), jnp.float32)]
    if with_next:
        out_specs.append(_col(n, h))
        outs.append(jax.ShapeDtypeStruct((n, h), jnp.float32))
    return pl.pallas_call(
        body, grid=(n // BN,), in_specs=in_specs, out_specs=out_specs,
        out_shape=outs)


@functools.lru_cache(maxsize=None)
def _tc_head(h, c):
    def body(f0, f1, f2, wi, bi, wd0, bd0, wd1, bd1, wo, bo, out_ref):
        f = jnp.concatenate([f0[...], f1[...], f2[...]], axis=-1)
        z = jnp.maximum(
            jnp.dot(f, wi[...], preferred_element_type=jnp.float32)
            + bi[...], 0.0)
        z = jnp.maximum(
            jnp.dot(z, wd0[...], preferred_element_type=jnp.float32)
            + bd0[...], 0.0)
        z = jnp.maximum(
            jnp.dot(z, wd1[...], preferred_element_type=jnp.float32)
            + bd1[...], 0.0)
        out_ref[...] = (jnp.dot(z, wo[...],
                                preferred_element_type=jnp.float32) + bo[...])

    return pl.pallas_call(
        body, out_shape=jax.ShapeDtypeStruct((G, c), jnp.float32))


def kernel(x, edge_index, batch, edge_weight, W0, b0, W1, b1, W2, b2,
           gw0, gb0, ga0, gw1, gb1, ga1, gw2, gb2, ga2,
           Wi, bi, Wd0, bd0, Wd1, bd1, Wo, bo):
    n, d = x.shape
    h = W0.shape[1]
    e = edge_weight.shape[0]
    c = Wo.shape[1]
    src, dst = edge_index[0], edge_index[1]
    row = lambda v: v.reshape(1, -1)
    br = batch.reshape(n // BN, 1, BN)
    bc = batch.reshape(n, 1)

    degf = _sc_deg(n, e)(dst, edge_weight)
    degt = degf.reshape(NC, n).T
    dinv, hp0 = _tc_prep(n, d, h)(degt, x, W0)

    msg = _sc_msg(n, h, e)

    def layer(hp, b, gw, gb, ga, wn, hprev):
        acc = msg(hp, src, dst, edge_weight)
        a0, a1 = acc[:n], acc[n:]
        pre, ssum, cnt = _tc_post_a(n, h)(a0, a1, hp, dinv, row(b), br)
        ctr, vsum = _tc_post_b(n, h)(pre, ssum, cnt, row(ga), br, bc)
        args = [ctr, vsum, cnt, row(gw), row(gb), dinv, bc]
        if wn is not None:
            args.append(wn)
        if hprev is not None:
            args.append(hprev)
        return _tc_post_c(n, h, wn is not None, hprev is not None)(*args)

    h1, flat0, hp1 = layer(hp0, b0, gw0, gb0, ga0, W1, None)
    h2, flat1, hp2 = layer(hp1, b1, gw1, gb1, ga1, W2, h1)
    h3, flat2 = layer(hp2, b2, gw2, gb2, ga2, None, h2)

    return _tc_head(h, c)(flat0, flat1, flat2, Wi, row(bi), Wd0, row(bd0),
                          Wd1, row(bd1), Wo, row(bo))


# speedup vs baseline: 11.3494x; 1.3955x over previous
import functools

import jax
import jax.numpy as jnp
from jax import lax
from jax.experimental import pallas as pl
from jax.experimental.pallas import tpu as pltpu
from jax.experimental.pallas import tpu_sc as plsc

NC = 2
NS = 16
L = 16
G = 16


def _mesh():
    return plsc.VectorSubcoreMesh(core_axis_name="c", subcore_axis_name="s")


@functools.lru_cache(maxsize=None)
def _sc_deg(n, e):
    ept = e // (NC * NS)
    ck = 80
    nchunks = ept // ck
    big = (n + NS - 1) // NS
    big = ((big + 7) // 8) * 8
    last = n - big * (NS - 1)

    @functools.partial(
        pl.kernel,
        out_type=jax.ShapeDtypeStruct((NC * n,), jnp.float32),
        mesh=_mesh(),
        scratch_types=[
            pltpu.VMEM_SHARED((n,), jnp.float32),
            pltpu.VMEM((big,), jnp.float32),
            pltpu.VMEM((ck,), jnp.int32),
            pltpu.VMEM((ck,), jnp.float32),
            pltpu.SemaphoreType.DMA,
        ],
    )
    def k(dst_hbm, w_hbm, out_hbm, acc, zbuf, dst_v, w_v, sem):
        cid = lax.axis_index("c")
        sid = lax.axis_index("s")
        wid = cid * NS + sid

        def zb(i, _):
            zbuf[pl.ds(i * L, L)] = jnp.zeros((L,), jnp.float32)
            return 0
        lax.fori_loop(0, big // L, zb, 0)

        @pl.when(sid < NS - 1)
        def _():
            pltpu.sync_copy(zbuf, acc.at[pl.ds(sid * big, big)])

        @pl.when(sid == NS - 1)
        def _():
            pltpu.sync_copy(zbuf.at[pl.ds(0, last)],
                            acc.at[pl.ds((NS - 1) * big, last)])

        plsc.subcore_barrier()

        def body(i, _):
            base = wid * ept + i * ck
            pltpu.sync_copy(dst_hbm.at[pl.ds(base, ck)], dst_v)
            pltpu.sync_copy(w_hbm.at[pl.ds(base, ck)], w_v)
            pltpu.async_copy(w_v, acc.at[dst_v], sem, add=True).wait()
            return 0
        lax.fori_loop(0, nchunks, body, 0)

        plsc.subcore_barrier()

        @pl.when(sid < NS - 1)
        def _():
            pltpu.sync_copy(acc.at[pl.ds(sid * big, big)], zbuf)
            pltpu.sync_copy(zbuf, out_hbm.at[pl.ds(cid * n + sid * big, big)])

        @pl.when(sid == NS - 1)
        def _():
            pltpu.sync_copy(acc.at[pl.ds((NS - 1) * big, last)],
                            zbuf.at[pl.ds(0, last)])
            pltpu.sync_copy(zbuf.at[pl.ds(0, last)],
                            out_hbm.at[pl.ds(cid * n + (NS - 1) * big, last)])

    return k


@functools.lru_cache(maxsize=None)
def _sc_msg(n, h, e):
    ck = 16
    SB = 5
    ept = e // (NC * NS)
    nch = ept // ck
    nsup = nch // SB
    BZ = 320
    bigr = ((n // NS + 7) // 8) * 8
    lastr = n - bigr * (NS - 1)

    def _sub_range(sid_static_last):
        total = lastr if sid_static_last else bigr
        parts, off = [], 0
        while off < total:
            cz = min(BZ, total - off)
            parts.append((off, cz))
            off += cz
        return parts

    @functools.partial(
        pl.kernel,
        out_type=jax.ShapeDtypeStruct((NC * n, h), jnp.float32),
        mesh=_mesh(),
        scratch_types=[
            pltpu.VMEM_SHARED((n, h), jnp.float32),
            pltpu.VMEM((ept,), jnp.int32),
            pltpu.VMEM((ept,), jnp.float32),
            [pltpu.VMEM((ck,), jnp.int32)] * SB,
            pltpu.VMEM((SB, ck, h), jnp.float32),
            [pltpu.SemaphoreType.DMA] * SB,
            [pltpu.SemaphoreType.DMA] * SB,
        ],
    )
    def k(hp_hbm, src_hbm, dst_hbm, w_hbm, zer_hbm, out_hbm,
          acc, src_v, w_v, dst_s, rows_v, gsems, ssems):
        cid = lax.axis_index("c")
        sid = lax.axis_index("s")
        wid = cid * NS + sid
        ebase = wid * ept

        def gather_issue(j, q):
            pltpu.async_copy(dst_hbm.at[pl.ds(ebase + j * ck, ck)],
                             dst_s[q], gsems[q])
            pltpu.async_copy(hp_hbm.at[src_v.at[pl.ds(j * ck, ck)]],
                             rows_v.at[q], gsems[q])

        def gather_wait(q):
            pltpu.make_async_copy(dst_hbm.at[pl.ds(0, ck)], dst_s[q],
                                  gsems[q]).wait()
            pltpu.make_async_copy(hp_hbm.at[pl.ds(0, ck)], rows_v.at[q],
                                  gsems[q]).wait()

        def scatter_wait(q):
            pltpu.make_async_copy(rows_v.at[q], acc.at[pl.ds(0, ck)],
                                  ssems[q]).wait()

        def scale(q, j):
            wg = w_v[pl.ds(j * ck, L)]
            for r in range(ck):
                wsp = jnp.full((L,), wg[r], jnp.float32)
                for c in range(h // L):
                    sl = pl.ds(c * L, L)
                    rows_v[q, r, sl] = rows_v[q, r, sl] * wsp

        pltpu.sync_copy(src_hbm.at[pl.ds(ebase, ept)], src_v)
        pltpu.sync_copy(w_hbm.at[pl.ds(ebase, ept)], w_v)
        gather_issue(0, 0)
        gather_issue(1, 1)

        @pl.when(sid < NS - 1)
        def _():
            base = sid * bigr
            pltpu.sync_copy(zer_hbm.at[pl.ds(base, bigr)],
                            acc.at[pl.ds(base, bigr)])

        @pl.when(sid == NS - 1)
        def _():
            base = (NS - 1) * bigr
            pltpu.sync_copy(zer_hbm.at[pl.ds(base, lastr)],
                            acc.at[pl.ds(base, lastr)])

        plsc.subcore_barrier()

        def super_body(s, _):
            for jp in range(SB):
                j = s * SB + jp
                gather_wait(jp)
                scale(jp, j)
                pltpu.async_copy(rows_v.at[jp], acc.at[dst_s[jp]],
                                 ssems[jp], add=True)
                q = (jp + 2) % SB
                if jp < SB - 2:
                    @pl.when(s > 0)
                    def _():
                        scatter_wait(q)
                    gather_issue(j + 2, q)
                else:
                    @pl.when(s < nsup - 1)
                    def _():
                        scatter_wait(q)
                        gather_issue(j + 2, q)
            return 0
        lax.fori_loop(0, nsup, super_body, 0)

        for q in range(SB):
            scatter_wait(q)

        plsc.subcore_barrier()

        @pl.when(sid < NS - 1)
        def _():
            base = sid * bigr
            pltpu.sync_copy(acc.at[pl.ds(base, bigr)],
                            out_hbm.at[pl.ds(cid * n + base, bigr)])

        @pl.when(sid == NS - 1)
        def _():
            base = (NS - 1) * bigr
            pltpu.sync_copy(acc.at[pl.ds(base, lastr)],
                            out_hbm.at[pl.ds(cid * n + base, lastr)])

    return k


BN = 1000


def _col(n, w):
    return pl.BlockSpec((BN, w), lambda i: (i, 0))


def _full(a, b):
    return pl.BlockSpec((a, b), lambda i: (0, 0))


@functools.lru_cache(maxsize=None)
def _tc_prep(n, d, h):
    def body(degt_ref, x_ref, w0_ref, dinv_ref, hp_ref):
        deg = jnp.sum(degt_ref[...], axis=1, keepdims=True) + 1.0
        dinv = lax.rsqrt(deg)
        dinv_ref[...] = dinv
        hw = jnp.dot(x_ref[...], w0_ref[...],
                     preferred_element_type=jnp.float32)
        hp_ref[...] = hw * dinv

    return pl.pallas_call(
        body,
        grid=(n // BN,),
        in_specs=[_col(n, 2), _col(n, d), _full(d, h)],
        out_specs=[_col(n, 1), _col(n, h)],
        out_shape=[jax.ShapeDtypeStruct((n, 1), jnp.float32),
                   jax.ShapeDtypeStruct((n, h), jnp.float32)],
    )


@functools.lru_cache(maxsize=None)
def _tc_post_a(n, h):
    def body(a0_ref, a1_ref, hp_ref, dinv_ref, b_ref, br_ref,
             pre_ref, ssum_ref, cnt_ref):
        i = pl.program_id(0)
        pre = ((a0_ref[...] + a1_ref[...] + hp_ref[...]) * dinv_ref[...]
               + b_ref[...])
        pre_ref[...] = pre
        oh = jnp.where(
            lax.broadcasted_iota(jnp.int32, (G, BN), 0) == br_ref[0],
            1.0, 0.0)

        @pl.when(i == 0)
        def _():
            ssum_ref[...] = jnp.zeros_like(ssum_ref)
            cnt_ref[...] = jnp.zeros_like(cnt_ref)

        ssum_ref[...] += jnp.dot(oh, pre, preferred_element_type=jnp.float32)
        cnt_ref[...] += jnp.sum(oh, axis=1, keepdims=True)

    return pl.pallas_call(
        body,
        grid=(n // BN,),
        in_specs=[_col(n, h), _col(n, h), _col(n, h), _col(n, 1),
                  _full(1, h), pl.BlockSpec((1, 1, BN), lambda i: (i, 0, 0))],
        out_specs=[_col(n, h), _full(G, h), _full(G, 1)],
        out_shape=[jax.ShapeDtypeStruct((n, h), jnp.float32),
                   jax.ShapeDtypeStruct((G, h), jnp.float32),
                   jax.ShapeDtypeStruct((G, 1), jnp.float32)],
    )


@functools.lru_cache(maxsize=None)
def _tc_post_b(n, h):
    def body(pre_ref, ssum_ref, cnt_ref, ga_ref, br_ref, bc_ref,
             ctr_ref, vsum_ref):
        i = pl.program_id(0)
        cnt = jnp.maximum(cnt_ref[...], 1.0)
        mean = ssum_ref[...] / cnt
        ohT = jnp.where(
            lax.broadcasted_iota(jnp.int32, (BN, G), 1) == bc_ref[...],
            1.0, 0.0)
        ctr = pre_ref[...] - ga_ref[...] * jnp.dot(
            ohT, mean, preferred_element_type=jnp.float32)
        ctr_ref[...] = ctr
        oh = jnp.where(
            lax.broadcasted_iota(jnp.int32, (G, BN), 0) == br_ref[0],
            1.0, 0.0)

        @pl.when(i == 0)
        def _():
            vsum_ref[...] = jnp.zeros_like(vsum_ref)

        vsum_ref[...] += jnp.dot(oh, ctr * ctr,
                                 preferred_element_type=jnp.float32)

    return pl.pallas_call(
        body,
        grid=(n // BN,),
        in_specs=[_col(n, h), _full(G, h), _full(G, 1), _full(1, h),
                  pl.BlockSpec((1, 1, BN), lambda i: (i, 0, 0)), _col(n, 1)],
        out_specs=[_col(n, h), _full(G, h)],
        out_shape=[jax.ShapeDtypeStruct((n, h), jnp.float32),
                   jax.ShapeDtypeStruct((G, h), jnp.float32)],
    )


@functools.lru_cache(maxsize=None)
def _tc_post_c(n, h, with_next, with_res):
    def body(*refs):
        it = iter(refs)
        ctr_ref = next(it)
        vsum_ref = next(it)
        cnt_ref = next(it)
        gw_ref = next(it)
        gb_ref = next(it)
        dinv_ref = next(it)
        bc_ref = next(it)
        wn_ref = next(it) if with_next else None
        hprev_ref = next(it) if with_res else None
        h_ref = next(it)
        flat_ref = next(it)
        hpn_ref = next(it) if with_next else None

        i = pl.program_id(0)
        cnt = jnp.maximum(cnt_ref[...], 1.0)
        den = jnp.sqrt(vsum_ref[...] / cnt + 1e-5)
        bc = bc_ref[...]
        ohT = jnp.where(
            lax.broadcasted_iota(jnp.int32, (BN, G), 1) == bc, 1.0, 0.0)
        hnew = jnp.maximum(
            gw_ref[...] * ctr_ref[...]
            / jnp.dot(ohT, den, preferred_element_type=jnp.float32)
            + gb_ref[...], 0.0)
        if with_res:
            hnew = hnew + hprev_ref[...]
        h_ref[...] = hnew

        neg = jnp.float32(-jnp.inf)
        rows = []
        for gg in range(G):
            rows.append(jnp.max(jnp.where(bc == gg, hnew, neg), axis=0,
                                keepdims=True))
        bmax = jnp.concatenate(rows, axis=0)

        @pl.when(i == 0)
        def _():
            flat_ref[...] = jnp.full_like(flat_ref, neg)

        flat_ref[...] = jnp.maximum(flat_ref[...], bmax)

        if with_next:
            hpn_ref[...] = jnp.dot(
                hnew, wn_ref[...],
                preferred_element_type=jnp.float32) * dinv_ref[...]

    in_specs = [_col(n, h), _full(G, h), _full(G, 1), _full(1, h),
                _full(1, h), _col(n, 1), _col(n, 1)]
    if with_next:
        in_specs.append(_full(h, h))
    if with_res:
        in_specs.append(_col(n, h))
    out_specs = [_col(n, h), _full(G, h)]
    outs = [jax.ShapeDtypeStruct((n, h), jnp.float32),
            jax.ShapeDtypeStruct((G, h), jnp.float32)]
    if with_next:
        out_specs.append(_col(n, h))
        outs.append(jax.ShapeDtypeStruct((n, h), jnp.float32))
    return pl.pallas_call(
        body, grid=(n // BN,), in_specs=in_specs, out_specs=out_specs,
        out_shape=outs)


@functools.lru_cache(maxsize=None)
def _tc_head(h, c):
    def body(f0, f1, f2, wi, bi, wd0, bd0, wd1, bd1, wo, bo, out_ref):
        f = jnp.concatenate([f0[...], f1[...], f2[...]], axis=-1)
        z = jnp.maximum(
            jnp.dot(f, wi[...], preferred_element_type=jnp.float32)
            + bi[...], 0.0)
        z = jnp.maximum(
            jnp.dot(z, wd0[...], preferred_element_type=jnp.float32)
            + bd0[...], 0.0)
        z = jnp.maximum(
            jnp.dot(z, wd1[...], preferred_element_type=jnp.float32)
            + bd1[...], 0.0)
        out_ref[...] = (jnp.dot(z, wo[...],
                                preferred_element_type=jnp.float32) + bo[...])

    return pl.pallas_call(
        body, out_shape=jax.ShapeDtypeStruct((G, c), jnp.float32))


def kernel(x, edge_index, batch, edge_weight, W0, b0, W1, b1, W2, b2,
           gw0, gb0, ga0, gw1, gb1, ga1, gw2, gb2, ga2,
           Wi, bi, Wd0, bd0, Wd1, bd1, Wo, bo):
    n, d = x.shape
    h = W0.shape[1]
    e = edge_weight.shape[0]
    c = Wo.shape[1]
    src, dst = edge_index[0], edge_index[1]
    row = lambda v: v.reshape(1, -1)
    br = batch.reshape(n // BN, 1, BN)
    bc = batch.reshape(n, 1)

    degf = _sc_deg(n, e)(dst, edge_weight)
    degt = degf.reshape(NC, n).T
    dinv, hp0 = _tc_prep(n, d, h)(degt, x, W0)

    msg = _sc_msg(n, h, e)

    zer = jnp.zeros((n, h), jnp.float32)

    def layer(hp, b, gw, gb, ga, wn, hprev):
        acc = msg(hp, src, dst, edge_weight, zer)
        a0, a1 = acc[:n], acc[n:]
        pre, ssum, cnt = _tc_post_a(n, h)(a0, a1, hp, dinv, row(b), br)
        ctr, vsum = _tc_post_b(n, h)(pre, ssum, cnt, row(ga), br, bc)
        args = [ctr, vsum, cnt, row(gw), row(gb), dinv, bc]
        if wn is not None:
            args.append(wn)
        if hprev is not None:
            args.append(hprev)
        return _tc_post_c(n, h, wn is not None, hprev is not None)(*args)

    h1, flat0, hp1 = layer(hp0, b0, gw0, gb0, ga0, W1, None)
    h2, flat1, hp2 = layer(hp1, b1, gw1, gb1, ga1, W2, h1)
    h3, flat2 = layer(hp2, b2, gw2, gb2, ga2, None, h2)

    return _tc_head(h, c)(flat0, flat1, flat2, Wi, row(bi), Wd0, row(bd0),
                          Wd1, row(bd1), Wo, row(bo))
